# baseline (speedup 1.0000x reference)
import jax
import jax.numpy as jnp
from jax import lax
from jax.experimental import pallas as pl
from jax.experimental.pallas import tpu as pltpu
from jax.experimental.pallas import tpu_sc as plsc

B = 4096
T = 77
D = 128
NC = 2
NS = 16
NW = NC * NS
NBUF = 8
TPAD = 80
CHUNKS = 2
CB = B // CHUNKS


def _make_gather(nb):
  seq_per_w = nb // NW

  def body(tok_hbm, table_hbm, out_hbm, idx_v, bufs, *sems):
    sem_g = sems[:NBUF]
    sem_s = sems[NBUF:]
    wid = lax.axis_index("s") * NC + lax.axis_index("c")
    seq0 = wid * seq_per_w

    pltpu.sync_copy(tok_hbm.at[pl.ds(seq0, seq_per_w)], idx_v)

    def gather(s, b):
      return pltpu.make_async_copy(table_hbm.at[idx_v.at[s]],
                                   bufs.at[b, pl.ds(0, T)], sem_g[b])

    def scatter(s, b):
      return pltpu.make_async_copy(
          bufs.at[b], out_hbm.at[pl.ds((seq0 + s) * TPAD, TPAD)], sem_s[b])

    def step(s, b, refill, drain):
      b2 = (b + NBUF // 2) % NBUF
      if refill:
        if drain:
          scatter(s - NBUF // 2, b2).wait()
        gather(s + NBUF // 2, b2).start()
      gather(s, b).wait()
      scatter(s, b).start()

    for b in range(NBUF // 2):
      gather(b, b).start()

    for b in range(NBUF):
      step(b, b, refill=True, drain=(b >= NBUF // 2))

    def outer(g, carry):
      for b in range(NBUF):
        step(g * NBUF + b, b, refill=True, drain=True)
      return carry

    lax.fori_loop(1, seq_per_w // NBUF - 1, outer, 0)

    g = seq_per_w // NBUF - 1
    for b in range(NBUF):
      step(g * NBUF + b, b, refill=(b < NBUF // 2), drain=(b < NBUF // 2))

    for b in range(NBUF):
      scatter(g * NBUF + b, b).wait()

  return pl.kernel(
      body,
      out_type=jax.ShapeDtypeStruct((nb * TPAD, D), jnp.float32),
      mesh=plsc.VectorSubcoreMesh(core_axis_name="c", subcore_axis_name="s"),
      scratch_types=[
          pltpu.VMEM((seq_per_w, T), jnp.int32),
          pltpu.VMEM((NBUF, TPAD, D), jnp.float32),
      ] + [pltpu.SemaphoreType.DMA] * (2 * NBUF),
  )


_gather_chunk = _make_gather(CB)


@jax.jit
def kernel(tokens, token_embedding, position_embedding):
  posb = position_embedding[None, :, :]
  out = lax.optimization_barrier(jnp.zeros((B, T, D), jnp.float32))
  for c in range(CHUNKS):
    tok_c = lax.slice_in_dim(tokens, c * CB, (c + 1) * CB, axis=0)
    g3 = _gather_chunk(tok_c, token_embedding).reshape(CB, TPAD, D)
    out = lax.dynamic_update_slice(out, g3[:, :T, :] + posb, (c * CB, 0, 0))
  return out


# speedup vs baseline: 1.2221x; 1.2221x over previous
import jax
import jax.numpy as jnp
from jax import lax
from jax.experimental import pallas as pl
from jax.experimental.pallas import tpu as pltpu
from jax.experimental.pallas import tpu_sc as plsc

B = 4096
T = 77
D = 128
NC = 2
NS = 16
NW = NC * NS
NBUF = 8
TPAD = 80
CHUNKS = 1
CB = B // CHUNKS


def _make_gather(nb):
  seq_per_w = nb // NW

  def body(tok_hbm, table_hbm, out_hbm, idx_v, bufs, *sems):
    sem_g = sems[:NBUF]
    sem_s = sems[NBUF:]
    wid = lax.axis_index("s") * NC + lax.axis_index("c")
    seq0 = wid * seq_per_w

    pltpu.sync_copy(tok_hbm.at[pl.ds(seq0, seq_per_w)], idx_v)

    def gather(s, b):
      return pltpu.make_async_copy(table_hbm.at[idx_v.at[s]],
                                   bufs.at[b, pl.ds(0, T)], sem_g[b])

    def scatter(s, b):
      return pltpu.make_async_copy(
          bufs.at[b], out_hbm.at[pl.ds((seq0 + s) * TPAD, TPAD)], sem_s[b])

    def step(s, b, refill, drain):
      b2 = (b + NBUF // 2) % NBUF
      if refill:
        if drain:
          scatter(s - NBUF // 2, b2).wait()
        gather(s + NBUF // 2, b2).start()
      gather(s, b).wait()
      scatter(s, b).start()

    for b in range(NBUF // 2):
      gather(b, b).start()

    for b in range(NBUF):
      step(b, b, refill=True, drain=(b >= NBUF // 2))

    def outer(g, carry):
      for b in range(NBUF):
        step(g * NBUF + b, b, refill=True, drain=True)
      return carry

    lax.fori_loop(1, seq_per_w // NBUF - 1, outer, 0)

    g = seq_per_w // NBUF - 1
    for b in range(NBUF):
      step(g * NBUF + b, b, refill=(b < NBUF // 2), drain=(b < NBUF // 2))

    for b in range(NBUF):
      scatter(g * NBUF + b, b).wait()

  return pl.kernel(
      body,
      out_type=jax.ShapeDtypeStruct((nb * TPAD, D), jnp.float32),
      mesh=plsc.VectorSubcoreMesh(core_axis_name="c", subcore_axis_name="s"),
      scratch_types=[
          pltpu.VMEM((seq_per_w, T), jnp.int32),
          pltpu.VMEM((NBUF, TPAD, D), jnp.float32),
      ] + [pltpu.SemaphoreType.DMA] * (2 * NBUF),
  )


_gather_chunk = _make_gather(CB)


@jax.jit
def kernel(tokens, token_embedding, position_embedding):
  posb = position_embedding[None, :, :]
  parts = []
  for c in range(CHUNKS):
    tok_c = lax.slice_in_dim(tokens, c * CB, (c + 1) * CB, axis=0)
    g3 = _gather_chunk(tok_c, token_embedding).reshape(CB, TPAD, D)
    parts.append(g3[:, :T, :] + posb)
  return jnp.concatenate(parts, axis=0)


# speedup vs baseline: 1.2233x; 1.0010x over previous
import jax
import jax.numpy as jnp
from jax import lax
from jax.experimental import pallas as pl
from jax.experimental.pallas import tpu as pltpu
from jax.experimental.pallas import tpu_sc as plsc

B = 4096
T = 77
D = 128
NC = 2
NS = 16
NW = NC * NS
NBUF = 8
TPAD = 80


def _make_gather(nb):
  seq_per_w = nb // NW

  def body(tok_hbm, table_hbm, out_hbm, idx_v, bufs, *sems):
    sem_g = sems[:NBUF]
    sem_s = sems[NBUF:]
    wid = lax.axis_index("s") * NC + lax.axis_index("c")
    seq0 = wid * seq_per_w

    pltpu.sync_copy(tok_hbm.at[pl.ds(seq0, seq_per_w)], idx_v)

    def gather(s, b):
      return pltpu.make_async_copy(table_hbm.at[idx_v.at[s]],
                                   bufs.at[b, pl.ds(0, T)], sem_g[b])

    def scatter(s, b):
      return pltpu.make_async_copy(
          bufs.at[b], out_hbm.at[pl.ds((seq0 + s) * TPAD, TPAD)], sem_s[b])

    def step(s, b, refill, drain):
      b2 = (b + NBUF // 2) % NBUF
      if refill:
        if drain:
          scatter(s - NBUF // 2, b2).wait()
        gather(s + NBUF // 2, b2).start()
      gather(s, b).wait()
      scatter(s, b).start()

    for b in range(NBUF // 2):
      gather(b, b).start()

    for b in range(NBUF):
      step(b, b, refill=True, drain=(b >= NBUF // 2))

    def outer(g, carry):
      for b in range(NBUF):
        step(g * NBUF + b, b, refill=True, drain=True)
      return carry

    lax.fori_loop(1, seq_per_w // NBUF - 1, outer, 0)

    g = seq_per_w // NBUF - 1
    for b in range(NBUF):
      step(g * NBUF + b, b, refill=(b < NBUF // 2), drain=(b < NBUF // 2))

    for b in range(NBUF):
      scatter(g * NBUF + b, b).wait()

  return pl.kernel(
      body,
      out_type=jax.ShapeDtypeStruct((nb * TPAD, D), jnp.float32),
      mesh=plsc.VectorSubcoreMesh(core_axis_name="c", subcore_axis_name="s"),
      scratch_types=[
          pltpu.VMEM((seq_per_w, T), jnp.int32),
          pltpu.VMEM((NBUF, TPAD, D), jnp.float32),
      ] + [pltpu.SemaphoreType.DMA] * (2 * NBUF),
  )


_gather = _make_gather(B)


@jax.jit
def kernel(tokens, token_embedding, position_embedding):
  g3 = _gather(tokens, token_embedding).reshape(B, TPAD, D)
  return g3[:, :T, :] + position_embedding[None, :, :]
